# baseline (speedup 1.0000x reference)
import functools

import jax
import jax.numpy as jnp
from jax import lax
from jax.experimental import pallas as pl
from jax.experimental.pallas import tpu as pltpu
from jax.experimental.pallas import tpu_sc as plsc

D = 1024
EPS = 1e-5
NW = 32
G = 16
R = 8
L = 16
NL = D // L


def _lane_sum(x):
    dnums = lax.GatherDimensionNumbers(
        offset_dims=(), collapsed_slice_dims=(0,), start_index_map=(0,))
    for sh in (8, 4, 2, 1):
        perm = lax.iota(jnp.int32, L) ^ sh
        x = x + lax.gather(x, perm[:, None], dnums, (1,),
                           mode=lax.GatherScatterMode.PROMISE_IN_BOUNDS)
    return x


def _rsqrt(x):
    bits = plsc.bitcast(x, jnp.int32)
    bits = jnp.int32(0x5F3759DF) - (bits >> 1)
    y = plsc.bitcast(bits, jnp.float32)
    for _ in range(2):
        y = y * (1.5 - 0.5 * x * y * y)
    return y


def _body(idx_hbm, tok_hbm, pos_hbm, gam_hbm, bet_hbm, out_hbm,
          idx_v, tbuf, pbuf, sbuf, gam_v, bet_v, gsem, osem,
          *, nch, seq, spw):
    nc = 2
    wid = lax.axis_index("s") * nc + lax.axis_index("c")
    jpb = spw // G

    pltpu.sync_copy(idx_hbm.at[wid], idx_v)
    pltpu.sync_copy(pos_hbm.at[pl.ds(wid * spw, spw)], pbuf)
    pltpu.sync_copy(gam_hbm, gam_v)
    pltpu.sync_copy(bet_hbm, bet_v)

    def out_base(k):
        return (k // jpb) * seq + wid * spw + (k % jpb) * G

    def start_fetch(k, slot):
        pltpu.async_copy(tok_hbm.at[idx_v.at[k]], tbuf.at[slot], gsem)

    def wait_fetch(k, slot):
        pltpu.make_async_copy(tok_hbm.at[idx_v.at[k]], tbuf.at[slot],
                              gsem).wait()

    def start_out(k, slot):
        pltpu.async_copy(tbuf.at[slot], out_hbm.at[pl.ds(out_base(k), G)],
                         osem)

    def wait_out(k, slot):
        pltpu.make_async_copy(tbuf.at[slot],
                              out_hbm.at[pl.ds(out_base(k), G)], osem).wait()

    start_fetch(0, 0)

    def chunk_body(k, carry):
        slot = k % 2
        other = 1 - slot
        p0 = (k % jpb) * G
        with jax.named_scope("wait_dma"):
            wait_fetch(k, slot)

            @pl.when(k >= 1)
            def _():
                wait_out(k - 1, other)

            @pl.when(k + 1 < nch)
            def _():
                start_fetch(k + 1, other)

        def strip_body(t, scarry):
            r0 = t * R
            sq = []
            scope_p1 = jax.named_scope("p1")
            scope_p1.__enter__()
            for rr in range(R):
                r = r0 + rr
                pr = p0 + r
                init = tuple(jnp.zeros((L,), jnp.float32) for _ in range(8))

                def p1_body(i, acc, *, _r=r, _pr=pr, _rr=rr):
                    vs = []
                    for j in range(4):
                        sl = pl.ds((i + j) * L, L)
                        v = tbuf[slot, _r, sl] + pbuf[_pr, sl]
                        sbuf[_rr, sl] = v
                        vs.append(v)
                    return (acc[0] + vs[0], acc[1] + vs[1],
                            acc[2] + vs[2], acc[3] + vs[3],
                            acc[4] + vs[0] * vs[0], acc[5] + vs[1] * vs[1],
                            acc[6] + vs[2] * vs[2], acc[7] + vs[3] * vs[3])

                a = plsc.parallel_loop(0, NL, 4, carry=init)(p1_body)
                sq.append(((a[0] + a[1]) + (a[2] + a[3]),
                           (a[4] + a[5]) + (a[6] + a[7])))

            scope_p1.__exit__(None, None, None)
            stats = []
            with jax.named_scope("stats"):
                for rr in range(R):
                    s, q = sq[rr]
                    mean = _lane_sum(s) * (1.0 / D)
                    var = _lane_sum(q) * (1.0 / D) - mean * mean
                    stats.append((mean, _rsqrt(var + EPS)))

            def p2_body(c):
                sl = pl.ds(c * L, L)
                g = gam_v[sl]
                b = bet_v[sl]
                for rr in range(R):
                    mean, rstd = stats[rr]
                    x = sbuf[rr, sl]
                    tbuf[slot, r0 + rr, sl] = (x - mean) * rstd * g + b

            with jax.named_scope("p2"):
                plsc.parallel_loop(0, NL, 1, unroll=2)(p2_body)
            return scarry

        lax.fori_loop(0, G // R, strip_body, 0)
        start_out(k, slot)
        return carry

    lax.fori_loop(0, nch, chunk_body, 0)
    wait_out(nch - 1, (nch - 1) % 2)


def kernel(x, token_table, pos_table, gamma, beta):
    b, s = x.shape
    n_tok = b * s
    spw = s // NW
    jpb = spw // G
    nch = b * jpb
    idx = (x.reshape(b, NW, jpb, G)
           .transpose(1, 0, 2, 3)
           .reshape(NW, nch, G)
           .astype(jnp.int32))

    mesh = plsc.VectorSubcoreMesh(core_axis_name="c", subcore_axis_name="s")
    run = pl.kernel(
        functools.partial(_body, nch=nch, seq=s, spw=spw),
        out_type=jax.ShapeDtypeStruct((n_tok, D), jnp.float32),
        mesh=mesh,
        compiler_params=pltpu.CompilerParams(needs_layout_passes=False),
        scratch_types=[
            pltpu.VMEM((nch, G), jnp.int32),
            pltpu.VMEM((2, G, D), jnp.float32),
            pltpu.VMEM((spw, D), jnp.float32),
            pltpu.VMEM((R, D), jnp.float32),
            pltpu.VMEM((D,), jnp.float32),
            pltpu.VMEM((D,), jnp.float32),
            pltpu.SemaphoreType.DMA,
            pltpu.SemaphoreType.DMA,
        ],
    )
    out = run(idx, token_table, pos_table, gamma, beta)
    return out.reshape(b, s, D)


# speedup vs baseline: 1.3921x; 1.3921x over previous
import functools

import jax
import jax.numpy as jnp
from jax import lax
from jax.experimental import pallas as pl
from jax.experimental.pallas import tpu as pltpu
from jax.experimental.pallas import tpu_sc as plsc

D = 1024
EPS = 1e-5
NW = 32
G = 16
R = 8
NB = 3
L = 16
NL = D // L


def _lane_sum(x):
    dnums = lax.GatherDimensionNumbers(
        offset_dims=(), collapsed_slice_dims=(0,), start_index_map=(0,))
    for sh in (8, 4, 2, 1):
        perm = lax.iota(jnp.int32, L) ^ sh
        x = x + lax.gather(x, perm[:, None], dnums, (1,),
                           mode=lax.GatherScatterMode.PROMISE_IN_BOUNDS)
    return x


def _rsqrt(x):
    bits = plsc.bitcast(x, jnp.int32)
    bits = jnp.int32(0x5F3759DF) - (bits >> 1)
    y = plsc.bitcast(bits, jnp.float32)
    for _ in range(2):
        y = y * (1.5 - 0.5 * x * y * y)
    return y


def _body(idx_hbm, tok_hbm, pos_hbm, gam_hbm, bet_hbm, out_hbm,
          idx_v, tbuf, pbuf, sbuf, gam_v, bet_v, psem, gsem, osem,
          *, nch, seq):
    nc = 2
    wid = lax.axis_index("s") * nc + lax.axis_index("c")
    tpw = nch * G
    base = wid * tpw
    s_off = (wid % (seq // tpw)) * tpw

    pltpu.sync_copy(idx_hbm.at[wid], idx_v)
    pltpu.sync_copy(gam_hbm, gam_v)
    pltpu.sync_copy(bet_hbm, bet_v)

    def start_fetch(k, slot):
        pltpu.async_copy(pos_hbm.at[pl.ds(s_off + k * G, G)],
                         pbuf.at[slot], psem)
        pltpu.async_copy(tok_hbm.at[idx_v.at[k]], tbuf.at[slot], gsem)

    def wait_fetch(k, slot):
        pltpu.make_async_copy(pos_hbm.at[pl.ds(s_off + k * G, G)],
                              pbuf.at[slot], psem).wait()
        pltpu.make_async_copy(tok_hbm.at[idx_v.at[k]], tbuf.at[slot],
                              gsem).wait()

    def start_out(k, slot):
        pltpu.async_copy(tbuf.at[slot], out_hbm.at[pl.ds(base + k * G, G)],
                         osem)

    def wait_out(k, slot):
        pltpu.make_async_copy(tbuf.at[slot],
                              out_hbm.at[pl.ds(base + k * G, G)], osem).wait()

    start_fetch(0, 0)
    start_fetch(1, 1)

    def chunk_body(k, carry):
        slot = k % NB

        with jax.named_scope("wait_dma"):
            wait_fetch(k, slot)

        def strip_body(t, scarry):
            r0 = t * R
            sq = []
            scope_p1 = jax.named_scope("p1")
            scope_p1.__enter__()
            for rr in range(R):
                r = r0 + rr
                init = tuple(jnp.zeros((L,), jnp.float32) for _ in range(8))

                def p1_body(i, acc, *, _r=r):
                    vs = []
                    for j in range(4):
                        sl = pl.ds((i + j) * L, L)
                        v = tbuf[slot, _r, sl] + pbuf[slot, _r, sl]
                        sbuf[_r, sl] = v
                        vs.append(v)
                    return (acc[0] + vs[0], acc[1] + vs[1],
                            acc[2] + vs[2], acc[3] + vs[3],
                            acc[4] + vs[0] * vs[0], acc[5] + vs[1] * vs[1],
                            acc[6] + vs[2] * vs[2], acc[7] + vs[3] * vs[3])

                a = plsc.parallel_loop(0, NL, 4, unroll=2,
                                       carry=init)(p1_body)
                sq.append(((a[0] + a[1]) + (a[2] + a[3]),
                           (a[4] + a[5]) + (a[6] + a[7])))
            scope_p1.__exit__(None, None, None)

            stats = []
            with jax.named_scope("stats"):
                for rr in range(R):
                    s, q = sq[rr]
                    mean = _lane_sum(s) * (1.0 / D)
                    var = _lane_sum(q) * (1.0 / D) - mean * mean
                    stats.append((mean, _rsqrt(var + EPS)))

            def p2_body(c):
                sl = pl.ds(c * L, L)
                g = gam_v[sl]
                b = bet_v[sl]
                for rr in range(R):
                    mean, rstd = stats[rr]
                    x = sbuf[r0 + rr, sl]
                    tbuf[slot, r0 + rr, sl] = (x - mean) * rstd * g + b

            with jax.named_scope("p2"):
                plsc.parallel_loop(0, NL, 1, unroll=2)(p2_body)
            return scarry

        lax.fori_loop(0, G // R, strip_body, 0)
        start_out(k, slot)

        with jax.named_scope("tail_dma"):
            @pl.when(k >= 1)
            def _():
                wait_out(k - 1, (k - 1) % NB)

            @pl.when(k + 2 < nch)
            def _():
                start_fetch(k + 2, (k + 2) % NB)

        return carry

    lax.fori_loop(0, nch, chunk_body, 0)
    wait_out(nch - 1, (nch - 1) % NB)


def kernel(x, token_table, pos_table, gamma, beta):
    b, s = x.shape
    n_tok = b * s
    tpw = n_tok // NW
    nch = tpw // G
    idx = x.reshape(NW, nch, G).astype(jnp.int32)

    mesh = plsc.VectorSubcoreMesh(core_axis_name="c", subcore_axis_name="s")
    run = pl.kernel(
        functools.partial(_body, nch=nch, seq=s),
        out_type=jax.ShapeDtypeStruct((n_tok, D), jnp.float32),
        mesh=mesh,
        compiler_params=pltpu.CompilerParams(needs_layout_passes=False),
        scratch_types=[
            pltpu.VMEM((nch, G), jnp.int32),
            pltpu.VMEM((NB, G, D), jnp.float32),
            pltpu.VMEM((NB, G, D), jnp.float32),
            pltpu.VMEM((G, D), jnp.float32),
            pltpu.VMEM((D,), jnp.float32),
            pltpu.VMEM((D,), jnp.float32),
            pltpu.SemaphoreType.DMA,
            pltpu.SemaphoreType.DMA,
            pltpu.SemaphoreType.DMA,
        ],
    )
    out = run(idx, token_table, pos_table, gamma, beta)
    return out.reshape(b, s, D)
